# baseline (speedup 1.0000x reference)
import functools

import jax
import jax.numpy as jnp
from jax.experimental import pallas as pl
from jax.experimental.pallas import tpu as pltpu

T = 2048
D_MODEL = 768
D_FF = 768
E = 8
BT = 256
FH = D_FF // 2


def _routing_kernel(x_ref, wg_ref, p_ref):
    logits = jnp.dot(x_ref[...], wg_ref[...], preferred_element_type=jnp.float32)
    eids = jax.lax.broadcasted_iota(jnp.int32, logits.shape, 1)
    m1 = jnp.max(logits, axis=-1, keepdims=True)
    i1 = jnp.min(jnp.where(logits == m1, eids, E), axis=-1, keepdims=True)
    rest = jnp.where(eids == i1, -jnp.inf, logits)
    m2 = jnp.max(rest, axis=-1, keepdims=True)
    i2 = jnp.min(jnp.where(rest == m2, eids, E), axis=-1, keepdims=True)
    w1 = jax.nn.sigmoid(m1 - m2)
    p_ref[...] = w1 * (eids == i1) + (1.0 - w1) * (eids == i2)


def _moe_kernel(p_ref, x_ref, wga_ref, wgb_ref, wua_ref, wub_ref,
                wda_ref, wdb_ref, out_ref):
    e = pl.program_id(0)

    @pl.when(e == 0)
    def _():
        out_ref[...] = x_ref[...]

    wga = wga_ref[0].astype(jnp.bfloat16)
    wgb = wgb_ref[0].astype(jnp.bfloat16)
    wua = wua_ref[0].astype(jnp.bfloat16)
    wub = wub_ref[0].astype(jnp.bfloat16)
    wda = wda_ref[0].astype(jnp.bfloat16)
    wdb = wdb_ref[0].astype(jnp.bfloat16)
    for tb in range(T // BT):
        rows = pl.ds(tb * BT, BT)
        xb16 = x_ref[rows, :].astype(jnp.bfloat16)
        hga = jnp.dot(xb16, wga, preferred_element_type=jnp.float32)
        hgb = jnp.dot(xb16, wgb, preferred_element_type=jnp.float32)
        hua = jnp.dot(xb16, wua, preferred_element_type=jnp.float32)
        hub = jnp.dot(xb16, wub, preferred_element_type=jnp.float32)
        ha = (hga * jax.nn.sigmoid(hga)) * hua
        hb = (hgb * jax.nn.sigmoid(hgb)) * hub
        y = (jnp.dot(ha.astype(jnp.bfloat16), wda,
                     preferred_element_type=jnp.float32)
             + jnp.dot(hb.astype(jnp.bfloat16), wdb,
                       preferred_element_type=jnp.float32))
        pb = p_ref[rows, :]
        eids = jax.lax.broadcasted_iota(jnp.int32, pb.shape, 1)
        w = jnp.sum(jnp.where(eids == e, pb, 0.0), axis=-1, keepdims=True)
        out_ref[rows, :] = out_ref[rows, :] + w * y


@jax.jit
def kernel(x, Wg, W_gate, W_up, W_down):
    P = pl.pallas_call(
        _routing_kernel,
        out_shape=jax.ShapeDtypeStruct((T, E), jnp.float32),
    )(x, Wg)

    y = pl.pallas_call(
        _moe_kernel,
        grid=(E,),
        in_specs=[
            pl.BlockSpec((T, E), lambda e: (0, 0)),
            pl.BlockSpec((T, D_MODEL), lambda e: (0, 0)),
            pl.BlockSpec((1, D_MODEL, FH), lambda e: (e, 0, 0)),
            pl.BlockSpec((1, D_MODEL, FH), lambda e: (e, 0, 1)),
            pl.BlockSpec((1, D_MODEL, FH), lambda e: (e, 0, 0)),
            pl.BlockSpec((1, D_MODEL, FH), lambda e: (e, 0, 1)),
            pl.BlockSpec((1, FH, D_MODEL), lambda e: (e, 0, 0)),
            pl.BlockSpec((1, FH, D_MODEL), lambda e: (e, 1, 0)),
        ],
        out_specs=pl.BlockSpec((T, D_MODEL), lambda e: (0, 0)),
        out_shape=jax.ShapeDtypeStruct((T, D_MODEL), jnp.float32),
    )(P, x, W_gate, W_gate, W_up, W_up, W_down, W_down)
    return y


# speedup vs baseline: 1.3671x; 1.3671x over previous
import functools

import jax
import jax.numpy as jnp
from jax.experimental import pallas as pl
from jax.experimental.pallas import tpu as pltpu

T = 2048
D_MODEL = 768
D_FF = 768
E = 8
BT = 256


def _routing_kernel(x_ref, wg_ref, p_ref):
    logits = jnp.dot(x_ref[...], wg_ref[...], preferred_element_type=jnp.float32)
    eids = jax.lax.broadcasted_iota(jnp.int32, logits.shape, 1)
    m1 = jnp.max(logits, axis=-1, keepdims=True)
    i1 = jnp.min(jnp.where(logits == m1, eids, E), axis=-1, keepdims=True)
    rest = jnp.where(eids == i1, -jnp.inf, logits)
    m2 = jnp.max(rest, axis=-1, keepdims=True)
    i2 = jnp.min(jnp.where(rest == m2, eids, E), axis=-1, keepdims=True)
    w1 = jax.nn.sigmoid(m1 - m2)
    p_ref[...] = w1 * (eids == i1) + (1.0 - w1) * (eids == i2)


def _moe_kernel(x_ref, wgate_ref, wg_ref, wu_ref, wd_ref, out_ref, p_ref):
    e = pl.program_id(0)

    @pl.when(e == 0)
    def _():
        _routing_kernel(x_ref, wgate_ref, p_ref)
        out_ref[...] = x_ref[...]

    wg16 = wg_ref[0].astype(jnp.bfloat16)
    wu16 = wu_ref[0].astype(jnp.bfloat16)
    wd16 = wd_ref[0].astype(jnp.bfloat16)
    for tb in range(T // BT):
        rows = pl.ds(tb * BT, BT)
        xb16 = x_ref[rows, :].astype(jnp.bfloat16)
        hg = jnp.dot(xb16, wg16, preferred_element_type=jnp.float32)
        hu = jnp.dot(xb16, wu16, preferred_element_type=jnp.float32)
        h = (hg * jax.nn.sigmoid(hg)) * hu
        y = jnp.dot(h.astype(jnp.bfloat16), wd16,
                    preferred_element_type=jnp.float32)
        pb = p_ref[rows, :]
        eids = jax.lax.broadcasted_iota(jnp.int32, pb.shape, 1)
        w = jnp.sum(jnp.where(eids == e, pb, 0.0), axis=-1, keepdims=True)
        out_ref[rows, :] = out_ref[rows, :] + w * y


@jax.jit
def kernel(x, Wg, W_gate, W_up, W_down):
    y = pl.pallas_call(
        _moe_kernel,
        grid=(E,),
        in_specs=[
            pl.BlockSpec((T, D_MODEL), lambda e: (0, 0)),
            pl.BlockSpec((D_MODEL, E), lambda e: (0, 0)),
            pl.BlockSpec((1, D_MODEL, D_FF), lambda e: (e, 0, 0)),
            pl.BlockSpec((1, D_MODEL, D_FF), lambda e: (e, 0, 0)),
            pl.BlockSpec((1, D_FF, D_MODEL), lambda e: (e, 0, 0)),
        ],
        out_specs=pl.BlockSpec((T, D_MODEL), lambda e: (0, 0)),
        out_shape=jax.ShapeDtypeStruct((T, D_MODEL), jnp.float32),
        scratch_shapes=[pltpu.VMEM((T, E), jnp.float32)],
    )(x, Wg, W_gate, W_up, W_down)
    return y
